# baseline (speedup 1.0000x reference)
import functools

import jax
import jax.numpy as jnp
from jax.experimental import pallas as pl
from jax.experimental.pallas import tpu as pltpu

K = 32


def _fold_max(x, n):
    for _ in range(n):
        h = x.shape[1] // 2
        x = jnp.maximum(x[:, :h], x[:, h:])
    return x


def _raise_to_k(x, t, c):
    def w_cond(st):
        return jnp.max(st[1]) > float(K)

    def w_body(st):
        t, c = st
        need = c > float(K)
        tn = jnp.min(jnp.where(x > t, x, jnp.inf), axis=1, keepdims=True)
        return (jnp.where(need, tn, t), jnp.where(need, c - 1.0, c))

    t, c = jax.lax.while_loop(w_cond, w_body, (t, c))
    return t


def _select_group_threshold(sb):
    s = sb[...]
    r = s.shape[0]
    s2 = _fold_max(s, 3)

    def body(_, carry):
        cur, _ = carry
        m = jnp.max(cur, axis=1, keepdims=True)
        cur = jnp.where(cur >= m, -jnp.inf, cur)
        return cur, m

    _, t2 = jax.lax.fori_loop(0, K, body,
                              (s2, jnp.zeros((r, 1), jnp.float32)))

    c_s = jnp.sum((s >= t2).astype(jnp.float32), axis=1, keepdims=True)
    return _raise_to_k(s, t2, c_s)


def _encode_topk_kernel(n_r, n_j, rb, jt, n_slices, A_ref, W_ref, acts_ref,
                        thr_ref, ab0, ab1, sb0, sb1, ts0, ts1):
    r = pl.program_id(0)
    j = pl.program_id(1)
    p = jax.lax.rem(r, 2)
    gt = jt // 8
    sl = rb // n_slices

    def compute_phase(ab, sb):
        blk = jnp.dot(A_ref[...], W_ref[...],
                      preferred_element_type=jnp.float32)
        acts_ref[...] = blk
        ab[:, pl.ds(j * jt, jt)] = blk
        sb[:, pl.ds(j * gt, gt)] = _fold_max(blk, 3)

    def select_s_phase(sb, ts):
        ts[...] = _select_group_threshold(sb)

    def select_acts_phase(ab, ts):
        rows = pl.ds((j - 1) * sl, sl)
        acts = ab[rows, :]
        t_s = ts[rows, :]
        c = jnp.sum((acts >= t_s).astype(jnp.float32), axis=1,
                    keepdims=True)
        t = _raise_to_k(acts, t_s, c)
        thr_ref[pl.ds((r - 1) * rb + (j - 1) * sl, sl), :] = t

    @pl.when(jnp.logical_and(r < n_r, p == 0))
    def _():
        compute_phase(ab0, sb0)

    @pl.when(jnp.logical_and(r < n_r, p == 1))
    def _():
        compute_phase(ab1, sb1)

    @pl.when(jnp.logical_and(jnp.logical_and(r >= 1, j == 0), p == 1))
    def _():
        select_s_phase(sb0, ts0)

    @pl.when(jnp.logical_and(jnp.logical_and(r >= 1, j == 0), p == 0))
    def _():
        select_s_phase(sb1, ts1)

    in_window = jnp.logical_and(j >= 1, j <= n_slices)

    @pl.when(jnp.logical_and(jnp.logical_and(r >= 1, in_window), p == 1))
    def _():
        select_acts_phase(ab0, ts0)

    @pl.when(jnp.logical_and(jnp.logical_and(r >= 1, in_window), p == 0))
    def _():
        select_acts_phase(ab1, ts1)


def _zdecode_kernel(b_ref, acts_ref, thr_ref, W_ref, out_ref, z_ref, *, rt):
    r = pl.program_id(0)
    t = pl.program_id(1)

    a = acts_ref[...]
    thr = thr_ref[pl.ds(r * rt, rt), :]
    zt = jnp.where(a >= thr, a, 0.0)
    z_ref[...] = zt

    @pl.when(t == 0)
    def _():
        out_ref[...] = jnp.broadcast_to(b_ref[...], out_ref.shape)

    out_ref[...] += jnp.dot(zt.astype(jnp.bfloat16), W_ref[...],
                            preferred_element_type=jnp.float32)


@jax.jit
def kernel(A, W_enc, W_dec, b_pre):
    n, d_act = A.shape
    d_hid = W_enc.shape[1]

    R = 256
    JT = 1024
    n_r = n // R
    n_j = d_hid // JT
    last = n_r - 1

    n_sl = 8
    while n_sl > n_j - 1:
        n_sl //= 2

    x = (A - b_pre).astype(jnp.bfloat16)
    W_bf = W_enc.astype(jnp.bfloat16)
    Wd_bf = W_dec.astype(jnp.bfloat16)
    b2 = b_pre.reshape(1, d_act)

    acts, thr = pl.pallas_call(
        functools.partial(_encode_topk_kernel, n_r, n_j, R, JT, n_sl),
        grid=(n_r + 1, n_j),
        in_specs=[
            pl.BlockSpec((R, d_act), lambda r, j: (jnp.minimum(r, last), 0)),
            pl.BlockSpec((d_act, JT), lambda r, j: (0, j)),
        ],
        out_specs=[
            pl.BlockSpec(
                (R, JT),
                lambda r, j: (jnp.minimum(r, last),
                              jnp.where(r < n_r, j, n_j - 1))),
            pl.BlockSpec((n, 1), lambda r, j: (0, 0)),
        ],
        out_shape=[
            jax.ShapeDtypeStruct((n, d_hid), jnp.float32),
            jax.ShapeDtypeStruct((n, 1), jnp.float32),
        ],
        scratch_shapes=[
            pltpu.VMEM((R, d_hid), jnp.float32),
            pltpu.VMEM((R, d_hid), jnp.float32),
            pltpu.VMEM((R, d_hid // 8), jnp.float32),
            pltpu.VMEM((R, d_hid // 8), jnp.float32),
            pltpu.VMEM((R, 1), jnp.float32),
            pltpu.VMEM((R, 1), jnp.float32),
        ],
        compiler_params=pltpu.CompilerParams(
            dimension_semantics=("arbitrary", "arbitrary"),
            vmem_limit_bytes=100 * 1024 * 1024,
        ),
    )(x, W_bf)

    RT = min(1024, n)
    KT = min(1024, d_hid)
    n_t = d_hid // KT

    out, z = pl.pallas_call(
        functools.partial(_zdecode_kernel, rt=RT),
        grid=(n // RT, n_t),
        in_specs=[
            pl.BlockSpec((1, d_act), lambda r, t: (0, 0)),
            pl.BlockSpec((RT, KT), lambda r, t: (r, t)),
            pl.BlockSpec((n, 1), lambda r, t: (0, 0)),
            pl.BlockSpec((KT, d_act), lambda r, t: (t, 0)),
        ],
        out_specs=[
            pl.BlockSpec((RT, d_act), lambda r, t: (r, 0)),
            pl.BlockSpec((RT, KT), lambda r, t: (r, t)),
        ],
        out_shape=[
            jax.ShapeDtypeStruct((n, d_act), jnp.float32),
            jax.ShapeDtypeStruct((n, d_hid), jnp.float32),
        ],
        compiler_params=pltpu.CompilerParams(
            dimension_semantics=("arbitrary", "arbitrary"),
            vmem_limit_bytes=100 * 1024 * 1024,
        ),
    )(b2, acts, thr, Wd_bf)

    return (out, acts, z)


# speedup vs baseline: 1.0676x; 1.0676x over previous
import functools

import jax
import jax.numpy as jnp
from jax.experimental import pallas as pl
from jax.experimental.pallas import tpu as pltpu

K = 32


def _fold_max(x, n):
    for _ in range(n):
        h = x.shape[1] // 2
        x = jnp.maximum(x[:, :h], x[:, h:])
    return x


def _raise_to_k(x, t, c):
    def w_cond(st):
        return jnp.max(st[1]) > float(K)

    def w_body(st):
        t, c = st
        need = c > float(K)
        tn = jnp.min(jnp.where(x > t, x, jnp.inf), axis=1, keepdims=True)
        return (jnp.where(need, tn, t), jnp.where(need, c - 1.0, c))

    t, c = jax.lax.while_loop(w_cond, w_body, (t, c))
    return t


def _select_group_threshold(sb):
    s = sb[...]
    r = s.shape[0]
    s2 = _fold_max(s, 3)

    def body(_, carry):
        cur, _ = carry
        m = jnp.max(cur, axis=1, keepdims=True)
        cur = jnp.where(cur >= m, -jnp.inf, cur)
        return cur, m

    _, t2 = jax.lax.fori_loop(0, K, body,
                              (s2, jnp.zeros((r, 1), jnp.float32)))

    c_s = jnp.sum((s >= t2).astype(jnp.float32), axis=1, keepdims=True)
    return _raise_to_k(s, t2, c_s)


def _encode_topk_kernel(n_r, n_j, rb, jt, n_slices, A_ref, W_ref, acts_ref,
                        thr_ref, ab0, ab1, sb, ts0, ts1):
    r = pl.program_id(0)
    j = pl.program_id(1)
    p = jax.lax.rem(r, 2)
    gt = jt // 16
    sl = rb // n_slices

    def compute_phase(ab):
        blk = jnp.dot(A_ref[...], W_ref[...],
                      preferred_element_type=jnp.float32)
        acts_ref[...] = blk
        ab[:, pl.ds(j * jt, jt)] = blk
        sb[:, pl.ds(j * gt, gt)] = _fold_max(blk, 4)

    def select_s_phase(ts):
        ts[...] = _select_group_threshold(sb)

    def select_acts_phase(ab, ts):
        rows = pl.ds((j - 1) * sl, sl)
        acts = ab[rows, :]
        t_s = ts[rows, :]
        c = jnp.sum((acts >= t_s).astype(jnp.float32), axis=1,
                    keepdims=True)
        t = _raise_to_k(acts, t_s, c)
        thr_ref[pl.ds((r - 1) * rb + (j - 1) * sl, sl), :] = t

    @pl.when(jnp.logical_and(r < n_r, p == 0))
    def _():
        compute_phase(ab0)

    @pl.when(jnp.logical_and(r < n_r, p == 1))
    def _():
        compute_phase(ab1)

    at_last = jnp.logical_and(r < n_r, j == n_j - 1)

    @pl.when(jnp.logical_and(at_last, p == 0))
    def _():
        select_s_phase(ts0)

    @pl.when(jnp.logical_and(at_last, p == 1))
    def _():
        select_s_phase(ts1)

    in_window = jnp.logical_and(
        r >= 1, jnp.logical_and(j >= 1, j <= n_slices))

    @pl.when(jnp.logical_and(in_window, p == 1))
    def _():
        select_acts_phase(ab0, ts0)

    @pl.when(jnp.logical_and(in_window, p == 0))
    def _():
        select_acts_phase(ab1, ts1)


def _zdecode_kernel(b_ref, acts_ref, thr_ref, W_ref, out_ref, z_ref, *, rt):
    r = pl.program_id(0)
    t = pl.program_id(1)

    a = acts_ref[...]
    thr = thr_ref[pl.ds(r * rt, rt), :]
    zt = jnp.where(a >= thr, a, 0.0)
    z_ref[...] = zt

    @pl.when(t == 0)
    def _():
        out_ref[...] = jnp.broadcast_to(b_ref[...], out_ref.shape)

    out_ref[...] += jnp.dot(zt.astype(jnp.bfloat16), W_ref[...],
                            preferred_element_type=jnp.float32)


@jax.jit
def kernel(A, W_enc, W_dec, b_pre):
    n, d_act = A.shape
    d_hid = W_enc.shape[1]

    R = 256
    JT = 2048
    n_r = n // R
    n_j = d_hid // JT
    last = n_r - 1

    n_sl = 8
    while n_sl > n_j - 1:
        n_sl //= 2

    x = (A - b_pre).astype(jnp.bfloat16)
    W_bf = W_enc.astype(jnp.bfloat16)
    Wd_bf = W_dec.astype(jnp.bfloat16)
    b2 = b_pre.reshape(1, d_act)

    acts, thr = pl.pallas_call(
        functools.partial(_encode_topk_kernel, n_r, n_j, R, JT, n_sl),
        grid=(n_r + 1, n_j),
        in_specs=[
            pl.BlockSpec((R, d_act), lambda r, j: (jnp.minimum(r, last), 0)),
            pl.BlockSpec((d_act, JT), lambda r, j: (0, j)),
        ],
        out_specs=[
            pl.BlockSpec(
                (R, JT),
                lambda r, j: (jnp.minimum(r, last),
                              jnp.where(r < n_r, j, n_j - 1))),
            pl.BlockSpec((n, 1), lambda r, j: (0, 0)),
        ],
        out_shape=[
            jax.ShapeDtypeStruct((n, d_hid), jnp.float32),
            jax.ShapeDtypeStruct((n, 1), jnp.float32),
        ],
        scratch_shapes=[
            pltpu.VMEM((R, d_hid), jnp.float32),
            pltpu.VMEM((R, d_hid), jnp.float32),
            pltpu.VMEM((R, d_hid // 16), jnp.float32),
            pltpu.VMEM((R, 1), jnp.float32),
            pltpu.VMEM((R, 1), jnp.float32),
        ],
        compiler_params=pltpu.CompilerParams(
            dimension_semantics=("arbitrary", "arbitrary"),
            vmem_limit_bytes=100 * 1024 * 1024,
        ),
    )(x, W_bf)

    RT = min(1024, n)
    KT = min(1024, d_hid)
    n_t = d_hid // KT

    out, z = pl.pallas_call(
        functools.partial(_zdecode_kernel, rt=RT),
        grid=(n // RT, n_t),
        in_specs=[
            pl.BlockSpec((1, d_act), lambda r, t: (0, 0)),
            pl.BlockSpec((RT, KT), lambda r, t: (r, t)),
            pl.BlockSpec((n, 1), lambda r, t: (0, 0)),
            pl.BlockSpec((KT, d_act), lambda r, t: (t, 0)),
        ],
        out_specs=[
            pl.BlockSpec((RT, d_act), lambda r, t: (r, 0)),
            pl.BlockSpec((RT, KT), lambda r, t: (r, t)),
        ],
        out_shape=[
            jax.ShapeDtypeStruct((n, d_act), jnp.float32),
            jax.ShapeDtypeStruct((n, d_hid), jnp.float32),
        ],
        compiler_params=pltpu.CompilerParams(
            dimension_semantics=("arbitrary", "arbitrary"),
            vmem_limit_bytes=100 * 1024 * 1024,
        ),
    )(b2, acts, thr, Wd_bf)

    return (out, acts, z)


# speedup vs baseline: 1.0722x; 1.0043x over previous
import functools

import jax
import jax.numpy as jnp
from jax.experimental import pallas as pl
from jax.experimental.pallas import tpu as pltpu

K = 32


def _fold_max(x, n):
    for _ in range(n):
        h = x.shape[1] // 2
        x = jnp.maximum(x[:, :h], x[:, h:])
    return x


def _raise_to_k(x, t, c):
    def w_cond(st):
        return jnp.max(st[1]) > float(K)

    def w_body(st):
        t, c = st
        need = c > float(K)
        tn = jnp.min(jnp.where(x > t, x, jnp.inf), axis=1, keepdims=True)
        return (jnp.where(need, tn, t), jnp.where(need, c - 1.0, c))

    t, c = jax.lax.while_loop(w_cond, w_body, (t, c))
    return t


def _select_group_threshold(sb):
    s = sb[...]
    r = s.shape[0]
    s2 = _fold_max(s, 3)

    def body(_, carry):
        cur, _ = carry
        m = jnp.max(cur, axis=1, keepdims=True)
        cur = jnp.where(cur >= m, -jnp.inf, cur)
        return cur, m

    _, t2 = jax.lax.fori_loop(0, K, body,
                              (s2, jnp.zeros((r, 1), jnp.float32)))

    c_s = jnp.sum((s >= t2).astype(jnp.float32), axis=1, keepdims=True)
    return _raise_to_k(s, t2, c_s)


def _encode_topk_kernel(n_r, n_j, rb, jt, n_slices, A_ref, W_ref, acts_ref,
                        thr_ref, ab0, ab1, sb, ts0, ts1):
    r = pl.program_id(0)
    j = pl.program_id(1)
    p = jax.lax.rem(r, 2)
    gt = jt // 16
    sl = rb // n_slices

    def compute_phase(ab):
        blk = jnp.dot(A_ref[...], W_ref[...],
                      preferred_element_type=jnp.float32)
        acts_ref[...] = blk
        ab[:, pl.ds(j * jt, jt)] = blk
        sb[:, pl.ds(j * gt, gt)] = _fold_max(blk, 4)

    def select_s_phase(ts):
        ts[...] = _select_group_threshold(sb)

    def select_acts_phase(ab, ts):
        rows = pl.ds((j - 1) * sl, sl)
        acts = ab[rows, :]
        t_s = ts[rows, :]
        c = jnp.sum((acts >= t_s).astype(jnp.float32), axis=1,
                    keepdims=True)
        t = _raise_to_k(acts, t_s, c)
        thr_ref[pl.ds((r - 1) * rb + (j - 1) * sl, sl), :] = t

    @pl.when(jnp.logical_and(r < n_r, p == 0))
    def _():
        compute_phase(ab0)

    @pl.when(jnp.logical_and(r < n_r, p == 1))
    def _():
        compute_phase(ab1)

    at_last = jnp.logical_and(r < n_r, j == n_j - 1)

    @pl.when(jnp.logical_and(at_last, p == 0))
    def _():
        select_s_phase(ts0)

    @pl.when(jnp.logical_and(at_last, p == 1))
    def _():
        select_s_phase(ts1)

    in_window = jnp.logical_and(
        r >= 1, jnp.logical_and(j >= 1, j <= n_slices))

    @pl.when(jnp.logical_and(in_window, p == 1))
    def _():
        select_acts_phase(ab0, ts0)

    @pl.when(jnp.logical_and(in_window, p == 0))
    def _():
        select_acts_phase(ab1, ts1)


def _zdecode_kernel(b_ref, acts_ref, thr_ref, W_ref, out_ref, z_ref, *, rt):
    r = pl.program_id(0)
    t = pl.program_id(1)

    a = acts_ref[...]
    thr = thr_ref[pl.ds(r * rt, rt), :]
    zt = jnp.where(a >= thr, a, 0.0)
    z_ref[...] = zt

    @pl.when(t == 0)
    def _():
        out_ref[...] = jnp.broadcast_to(b_ref[...], out_ref.shape)

    out_ref[...] += jnp.dot(zt.astype(jnp.bfloat16), W_ref[...],
                            preferred_element_type=jnp.float32)


@jax.jit
def kernel(A, W_enc, W_dec, b_pre):
    n, d_act = A.shape
    d_hid = W_enc.shape[1]

    R = 256
    JT = 2048
    n_r = n // R
    n_j = d_hid // JT
    last = n_r - 1

    n_sl = 8
    while n_sl > n_j - 1:
        n_sl //= 2

    x = (A - b_pre).astype(jnp.bfloat16)
    W_bf = W_enc.astype(jnp.bfloat16)
    Wd_bf = W_dec.astype(jnp.bfloat16)
    b2 = b_pre.reshape(1, d_act)

    acts, thr = pl.pallas_call(
        functools.partial(_encode_topk_kernel, n_r, n_j, R, JT, n_sl),
        grid=(n_r + 1, n_j),
        in_specs=[
            pl.BlockSpec((R, d_act), lambda r, j: (jnp.minimum(r, last), 0)),
            pl.BlockSpec((d_act, JT),
                         lambda r, j: (0, jnp.where(r < n_r, j, 0))),
        ],
        out_specs=[
            pl.BlockSpec(
                (R, JT),
                lambda r, j: (jnp.minimum(r, last),
                              jnp.where(r < n_r, j, n_j - 1))),
            pl.BlockSpec((n, 1), lambda r, j: (0, 0)),
        ],
        out_shape=[
            jax.ShapeDtypeStruct((n, d_hid), jnp.float32),
            jax.ShapeDtypeStruct((n, 1), jnp.float32),
        ],
        scratch_shapes=[
            pltpu.VMEM((R, d_hid), jnp.float32),
            pltpu.VMEM((R, d_hid), jnp.float32),
            pltpu.VMEM((R, d_hid // 16), jnp.float32),
            pltpu.VMEM((R, 1), jnp.float32),
            pltpu.VMEM((R, 1), jnp.float32),
        ],
        compiler_params=pltpu.CompilerParams(
            dimension_semantics=("arbitrary", "arbitrary"),
            vmem_limit_bytes=100 * 1024 * 1024,
        ),
    )(x, W_bf)

    RT = min(1024, n)
    KT = min(1024, d_hid)
    n_t = d_hid // KT

    out, z = pl.pallas_call(
        functools.partial(_zdecode_kernel, rt=RT),
        grid=(n // RT, n_t),
        in_specs=[
            pl.BlockSpec((1, d_act), lambda r, t: (0, 0)),
            pl.BlockSpec((RT, KT), lambda r, t: (r, t)),
            pl.BlockSpec((n, 1), lambda r, t: (0, 0)),
            pl.BlockSpec((KT, d_act), lambda r, t: (t, 0)),
        ],
        out_specs=[
            pl.BlockSpec((RT, d_act), lambda r, t: (r, 0)),
            pl.BlockSpec((RT, KT), lambda r, t: (r, t)),
        ],
        out_shape=[
            jax.ShapeDtypeStruct((n, d_act), jnp.float32),
            jax.ShapeDtypeStruct((n, d_hid), jnp.float32),
        ],
        compiler_params=pltpu.CompilerParams(
            dimension_semantics=("arbitrary", "arbitrary"),
            vmem_limit_bytes=100 * 1024 * 1024,
        ),
    )(b2, acts, thr, Wd_bf)

    return (out, acts, z)
